# baseline (speedup 1.0000x reference)
import functools

import jax
import jax.numpy as jnp
from jax import lax
from jax.experimental import pallas as pl
from jax.experimental.pallas import tpu as pltpu
from jax.experimental.pallas import tpu_sc as plsc

B = 16384
D = 64
K = 8
N = B * D
TAIL = 3.0
MIN_W = 0.001
MIN_H = 0.001
MIN_D = 0.001
CW = 2 * TAIL - K * MIN_W
CH = 2 * TAIL - K * MIN_H
LN2 = 0.6931471805599453
SQRTH = 1.4142135381698608

NW = 32
ROWS_PER_W = B // NW
BLK = 16
NBLK = ROWS_PER_W // BLK


def _plog(v):
    bits = plsc.bitcast(v, jnp.int32)
    e = lax.shift_right_logical(bits, 23) - 127
    m = plsc.bitcast((bits & 0x007FFFFF) | 0x3F800000, jnp.float32)
    big = m > SQRTH
    m = jnp.where(big, m * 0.5, m)
    e = e + jnp.where(big, 1, 0)
    r = m - 1.0
    z = r * r
    p = jnp.float32(7.0376836292e-2)
    for c in (-1.1514610310e-1, 1.1676998740e-1, -1.2420140846e-1,
              1.4249322787e-1, -1.6668057665e-1, 2.0000714765e-1,
              -2.4999993993e-1, 3.3333331174e-1):
        p = p * r + c
    y = r * z * p - 0.5 * z
    return r + y + e.astype(jnp.float32) * LN2


def _softplus(u):
    t = jnp.exp(-jnp.abs(u))
    return jnp.maximum(u, 0.0) + _plog(1.0 + t)


def _sc_body(x_hbm, uw_hbm, uh_hbm, ud_hbm, out_hbm, ld_hbm,
             xv, uwv, uhv, udv, outv, ldv):
    wid = lax.axis_index("s") * 2 + lax.axis_index("c")
    lane = lax.iota(jnp.int32, 16)

    def block_body(b, _):
        row0 = wid * ROWS_PER_W + b * BLK
        e0 = row0 * D
        pltpu.sync_copy(x_hbm.at[pl.ds(pl.multiple_of(e0, 1024), BLK * D)], xv)
        pltpu.sync_copy(uw_hbm.at[pl.ds(pl.multiple_of(e0 * K, 8192), BLK * D * K)], uwv)
        pltpu.sync_copy(uh_hbm.at[pl.ds(pl.multiple_of(e0 * K, 8192), BLK * D * K)], uhv)
        pltpu.sync_copy(ud_hbm.at[pl.ds(pl.multiple_of(e0 * (K + 1), 1024), BLK * D * (K + 1))], udv)

        def col_body(c, ld_acc):
            x = plsc.load_gather(xv, [lane * D + c])
            ubase = lane * (D * K) + c * K
            w = [plsc.load_gather(uwv, [ubase + k]) for k in range(K)]
            h = [plsc.load_gather(uhv, [ubase + k]) for k in range(K)]
            dbase = lane * (D * (K + 1)) + c * (K + 1)
            u = [plsc.load_gather(udv, [dbase + k]) for k in range(K + 1)]

            mw = w[0]
            for k in range(1, K):
                mw = jnp.maximum(mw, w[k])
            tw = [jnp.exp(w[k] - mw) for k in range(K)]
            sw = tw[0]
            for k in range(1, K):
                sw = sw + tw[k]
            fw = CW / sw
            widths = [MIN_W + tw[k] * fw for k in range(K)]
            kx = [jnp.full((16,), -TAIL, jnp.float32)]
            for k in range(K):
                kx.append(kx[k] + widths[k])

            mh = h[0]
            for k in range(1, K):
                mh = jnp.maximum(mh, h[k])
            th = [jnp.exp(h[k] - mh) for k in range(K)]
            sh = th[0]
            for k in range(1, K):
                sh = sh + th[k]
            fh = CH / sh
            heights = [MIN_H + th[k] * fh for k in range(K)]
            ky = [jnp.full((16,), -TAIL, jnp.float32)]
            for k in range(K):
                ky.append(ky[k] + heights[k])

            cnt = jnp.where(kx[0] <= x, 1, 0)
            for k in range(1, K):
                cnt = cnt + jnp.where(kx[k] <= x, 1, 0)
            bin_ = jnp.clip(cnt - 1, 0, K - 1)

            ge = [bin_ >= k for k in range(1, K + 1)]

            def ladder(vals, shift=0):
                r = vals[shift]
                for j in range(1, len(vals) - shift):
                    r = jnp.where(ge[j - 1], vals[j + shift], r)
                return r

            x_k = ladder(kx)
            y_k = ladder(ky)
            w_b = ladder(widths)
            h_b = ladder(heights)
            u_k = ladder(u)
            u_k1 = ladder(u, shift=1)
            d_k = MIN_D + _softplus(u_k)
            d_k1 = MIN_D + _softplus(u_k1)

            inv_w = 1.0 / w_b
            xi = (x - x_k) * inv_w
            s_k = h_b * inv_w
            omx = 1.0 - xi
            ximx = xi * omx
            xi2 = xi * xi
            num = h_b * (s_k * xi2 + d_k * ximx)
            den = s_k + (d_k1 + d_k - 2.0 * s_k) * ximx
            y = y_k + num / den
            num_g = s_k * s_k * (d_k1 * xi2 + 2.0 * s_k * ximx + d_k * omx * omx)
            dy_dx = num_g / (den * den) * inv_w
            log_det = _plog(jnp.abs(dy_dx))

            outside = (x <= -TAIL) | (x >= TAIL)
            y_out = jnp.where(outside, x, y)
            ld = jnp.where(outside, 0.0, log_det)
            plsc.store_scatter(outv, [lane * D + c], y_out)
            return ld_acc + ld

        ld_acc = lax.fori_loop(0, D, col_body, jnp.zeros((16,), jnp.float32))
        ldv[...] = ld_acc
        pltpu.sync_copy(outv, out_hbm.at[pl.ds(pl.multiple_of(e0, 1024), BLK * D)])
        pltpu.sync_copy(ldv, ld_hbm.at[pl.ds(pl.multiple_of(row0, 16), BLK)])
        return 0

    lax.fori_loop(0, NBLK, block_body, 0)


@jax.jit
def _run(x, uw, uh, ud):
    mesh = plsc.VectorSubcoreMesh(core_axis_name="c", subcore_axis_name="s")
    f = pl.kernel(
        _sc_body,
        mesh=mesh,
        compiler_params=pltpu.CompilerParams(needs_layout_passes=False),
        out_type=(
            jax.ShapeDtypeStruct((N,), jnp.float32),
            jax.ShapeDtypeStruct((B,), jnp.float32),
        ),
        scratch_types=[
            pltpu.VMEM((BLK * D,), jnp.float32),
            pltpu.VMEM((BLK * D * K,), jnp.float32),
            pltpu.VMEM((BLK * D * K,), jnp.float32),
            pltpu.VMEM((BLK * D * (K + 1),), jnp.float32),
            pltpu.VMEM((BLK * D,), jnp.float32),
            pltpu.VMEM((BLK,), jnp.float32),
        ],
    )
    out_flat, ld = f(x.reshape(N), uw.reshape(N * K), uh.reshape(N * K),
                     ud.reshape(N * (K + 1)))
    return out_flat.reshape(B, D), ld


def kernel(x, unnormalized_widths, unnormalized_heights, unnormalized_derivatives):
    return _run(x, unnormalized_widths, unnormalized_heights,
                unnormalized_derivatives)


# speedup vs baseline: 3.9527x; 3.9527x over previous
import functools

import jax
import jax.numpy as jnp
from jax import lax
from jax.experimental import pallas as pl
from jax.experimental.pallas import tpu as pltpu
from jax.experimental.pallas import tpu_sc as plsc

B = 16384
D = 64
K = 8
N = B * D
TAIL = 3.0
MIN_W = 0.001
MIN_H = 0.001
MIN_D = 0.001
CW = 2 * TAIL - K * MIN_W
CH = 2 * TAIL - K * MIN_H
LN2 = 0.6931471805599453
SQRTH = 1.4142135381698608

NW = 32
BW = B // NW
NG = BW // 16


def _plog(v):
    bits = plsc.bitcast(v, jnp.int32)
    e = lax.shift_right_logical(bits, 23) - 127
    m = plsc.bitcast((bits & 0x007FFFFF) | 0x3F800000, jnp.float32)
    big = m > SQRTH
    m = jnp.where(big, m * 0.5, m)
    e = e + jnp.where(big, 1, 0)
    r = m - 1.0
    z = r * r
    p = jnp.float32(7.0376836292e-2)
    for c in (-1.1514610310e-1, 1.1676998740e-1, -1.2420140846e-1,
              1.4249322787e-1, -1.6668057665e-1, 2.0000714765e-1,
              -2.4999993993e-1, 3.3333331174e-1):
        p = p * r + c
    y = r * z * p - 0.5 * z
    return r + y + e.astype(jnp.float32) * LN2


def _softplus(u):
    t = jnp.exp(-jnp.abs(u))
    return jnp.maximum(u, 0.0) + _plog(1.0 + t)


def _sc_body(x_hbm, uw_hbm, uh_hbm, ud_hbm, out_hbm, ld_hbm,
             xv, uwv, uhv, udv, outv, ldv):
    wid = lax.axis_index("s") * 2 + lax.axis_index("c")
    b0 = pl.multiple_of(wid * BW, BW)

    def zero_body(g, _):
        ldv[pl.ds(g * 16, 16)] = jnp.zeros((16,), jnp.float32)
        return 0

    lax.fori_loop(0, NG, zero_body, 0)

    def col_body(d, _):
        pltpu.sync_copy(x_hbm.at[d, pl.ds(b0, BW)], xv)
        pltpu.sync_copy(uw_hbm.at[d, :, pl.ds(b0, BW)], uwv)
        pltpu.sync_copy(uh_hbm.at[d, :, pl.ds(b0, BW)], uhv)
        pltpu.sync_copy(ud_hbm.at[:, d, pl.ds(b0, BW)], udv)

        def grp_body(g, _):
            sl = pl.ds(g * 16, 16)
            x = xv[sl]
            w = [uwv[k, sl] for k in range(K)]
            h = [uhv[k, sl] for k in range(K)]
            u = [udv[k, sl] for k in range(K + 1)]

            mw = w[0]
            for k in range(1, K):
                mw = jnp.maximum(mw, w[k])
            tw = [jnp.exp(w[k] - mw) for k in range(K)]
            sw = tw[0]
            for k in range(1, K):
                sw = sw + tw[k]
            fw = CW / sw
            widths = [MIN_W + tw[k] * fw for k in range(K)]
            kx = [jnp.full((16,), -TAIL, jnp.float32)]
            for k in range(K):
                kx.append(kx[k] + widths[k])

            mh = h[0]
            for k in range(1, K):
                mh = jnp.maximum(mh, h[k])
            th = [jnp.exp(h[k] - mh) for k in range(K)]
            sh = th[0]
            for k in range(1, K):
                sh = sh + th[k]
            fh = CH / sh
            heights = [MIN_H + th[k] * fh for k in range(K)]
            ky = [jnp.full((16,), -TAIL, jnp.float32)]
            for k in range(K):
                ky.append(ky[k] + heights[k])

            cnt = jnp.where(kx[0] <= x, 1, 0)
            for k in range(1, K):
                cnt = cnt + jnp.where(kx[k] <= x, 1, 0)
            bin_ = jnp.clip(cnt - 1, 0, K - 1)

            ge = [bin_ >= k for k in range(1, K + 1)]

            def ladder(vals, shift=0):
                r = vals[shift]
                for j in range(1, len(vals) - shift):
                    r = jnp.where(ge[j - 1], vals[j + shift], r)
                return r

            x_k = ladder(kx)
            y_k = ladder(ky)
            w_b = ladder(widths)
            h_b = ladder(heights)
            u_k = ladder(u)
            u_k1 = ladder(u, shift=1)
            d_k = MIN_D + _softplus(u_k)
            d_k1 = MIN_D + _softplus(u_k1)

            inv_w = 1.0 / w_b
            xi = (x - x_k) * inv_w
            s_k = h_b * inv_w
            omx = 1.0 - xi
            ximx = xi * omx
            xi2 = xi * xi
            num = h_b * (s_k * xi2 + d_k * ximx)
            den = s_k + (d_k1 + d_k - 2.0 * s_k) * ximx
            y = y_k + num / den
            num_g = s_k * s_k * (d_k1 * xi2 + 2.0 * s_k * ximx + d_k * omx * omx)
            dy_dx = num_g / (den * den) * inv_w
            log_det = _plog(jnp.abs(dy_dx))

            outside = (x <= -TAIL) | (x >= TAIL)
            outv[sl] = jnp.where(outside, x, y)
            ldv[sl] = ldv[sl] + jnp.where(outside, 0.0, log_det)
            return 0

        lax.fori_loop(0, NG, grp_body, 0)
        pltpu.sync_copy(outv, out_hbm.at[d, pl.ds(b0, BW)])
        return 0

    lax.fori_loop(0, D, col_body, 0)
    pltpu.sync_copy(ldv, ld_hbm.at[pl.ds(b0, BW)])


@jax.jit
def _run(x, uw, uh, ud):
    mesh = plsc.VectorSubcoreMesh(core_axis_name="c", subcore_axis_name="s")
    f = pl.kernel(
        _sc_body,
        mesh=mesh,
        compiler_params=pltpu.CompilerParams(needs_layout_passes=False),
        out_type=(
            jax.ShapeDtypeStruct((D, B), jnp.float32),
            jax.ShapeDtypeStruct((B,), jnp.float32),
        ),
        scratch_types=[
            pltpu.VMEM((BW,), jnp.float32),
            pltpu.VMEM((K, BW), jnp.float32),
            pltpu.VMEM((K, BW), jnp.float32),
            pltpu.VMEM((K + 1, BW), jnp.float32),
            pltpu.VMEM((BW,), jnp.float32),
            pltpu.VMEM((BW,), jnp.float32),
        ],
    )
    out_t, ld = f(x.T, uw.transpose(1, 2, 0), uh.transpose(1, 2, 0),
                  ud.transpose(2, 1, 0))
    return out_t.T, ld


def kernel(x, unnormalized_widths, unnormalized_heights, unnormalized_derivatives):
    return _run(x, unnormalized_widths, unnormalized_heights,
                unnormalized_derivatives)


# speedup vs baseline: 14.0413x; 3.5523x over previous
import functools

import jax
import jax.numpy as jnp
from jax import lax
from jax.experimental import pallas as pl
from jax.experimental.pallas import tpu as pltpu
from jax.experimental.pallas import tpu_sc as plsc

B = 16384
D = 64
K = 8
TAIL = 3.0
MIN_W = 0.001
MIN_H = 0.001
MIN_D = 0.001
CW = 2 * TAIL - K * MIN_W
CH = 2 * TAIL - K * MIN_H
LN2 = 0.6931471805599453
SQRTH = 1.4142135381698608
L1P = (0.010243828, -0.05326748, 0.13198966, -0.2239669,
       0.32751173, -0.49933395, 0.99997026, 2.2159765e-07)
PLOG = (-1.1514610310e-1, 1.1676998740e-1, -1.2420140846e-1,
        1.4249322787e-1, -1.6668057665e-1, 2.0000714765e-1,
        -2.4999993993e-1, 3.3333331174e-1)

NW = 32
BW = B // NW
NG = BW // 16


def _plog(v):
    bits = plsc.bitcast(v, jnp.int32)
    e = lax.shift_right_logical(bits, 23) - 127
    m = plsc.bitcast((bits & 0x007FFFFF) | 0x3F800000, jnp.float32)
    big = m > SQRTH
    m = jnp.where(big, m * 0.5, m)
    e = e + jnp.where(big, 1, 0)
    r = m - 1.0
    z = r * r
    p = jnp.float32(7.0376836292e-2)
    for c in PLOG:
        p = p * r + c
    return r + (r * z * p - 0.5 * z) + e.astype(jnp.float32) * LN2


def _softplus(u):
    t = jnp.exp(-jnp.abs(u))
    p = jnp.float32(L1P[0])
    for c in L1P[1:]:
        p = p * t + c
    return jnp.maximum(u, 0.0) + p


def _sc_body(x_hbm, uw_hbm, uh_hbm, ud_hbm, out_hbm, ld_hbm,
             xv, uwv, uhv, udv, outv, ldv,
             in_sem0, in_sem1, out_sem0, out_sem1):
    wid = lax.axis_index("s") * 2 + lax.axis_index("c")
    b0 = pl.multiple_of(wid * BW, BW)
    bsl = pl.ds(b0, BW)
    in_sems = (in_sem0, in_sem1)
    out_sems = (out_sem0, out_sem1)

    @plsc.parallel_loop(0, NG)
    def _zero(g):
        ldv[pl.ds(g * 16, 16)] = jnp.zeros((16,), jnp.float32)

    def issue_in(d, slot):
        sem = in_sems[slot]
        pltpu.async_copy(x_hbm.at[d, bsl], xv.at[slot], sem)
        pltpu.async_copy(uw_hbm.at[d, :, bsl], uwv.at[slot], sem)
        pltpu.async_copy(uh_hbm.at[d, :, bsl], uhv.at[slot], sem)
        pltpu.async_copy(ud_hbm.at[:, d, bsl], udv.at[slot], sem)

    def wait_in(d, slot):
        sem = in_sems[slot]
        pltpu.make_async_copy(x_hbm.at[d, bsl], xv.at[slot], sem).wait()
        pltpu.make_async_copy(uw_hbm.at[d, :, bsl], uwv.at[slot], sem).wait()
        pltpu.make_async_copy(uh_hbm.at[d, :, bsl], uhv.at[slot], sem).wait()
        pltpu.make_async_copy(ud_hbm.at[:, d, bsl], udv.at[slot], sem).wait()

    issue_in(0, 0)
    issue_in(1, 1)

    def process(d, slot):
        wait_in(d, slot)

        @pl.when(d >= 2)
        def _():
            pltpu.make_async_copy(outv.at[slot], out_hbm.at[d, bsl],
                                  out_sems[slot]).wait()

        @plsc.parallel_loop(0, NG)
        def _grp(g):
            sl = pl.ds(g * 16, 16)
            x = xv[slot, sl]
            tw = [jnp.exp(uwv[slot, k, sl]) for k in range(K)]
            th = [jnp.exp(uhv[slot, k, sl]) for k in range(K)]
            u = [udv[slot, k, sl] for k in range(K + 1)]

            sw = tw[0]
            for k in range(1, K):
                sw = sw + tw[k]
            fw = CW / sw
            cwk = [tw[k] * fw for k in range(K)]
            kx = [jnp.full((16,), -TAIL, jnp.float32)]
            for k in range(K):
                kx.append(kx[k] + (MIN_W + cwk[k]))

            sh = th[0]
            for k in range(1, K):
                sh = sh + th[k]
            fh = CH / sh
            chk = [th[k] * fh for k in range(K)]
            ky = [jnp.full((16,), -TAIL, jnp.float32)]
            for k in range(K):
                ky.append(ky[k] + (MIN_H + chk[k]))

            m = [kx[k] <= x for k in range(1, K)]

            def ladder(vals, shift=0):
                r = vals[shift]
                for j in range(1, len(vals) - shift):
                    r = jnp.where(m[j - 1], vals[j + shift], r)
                return r

            x_k = ladder(kx[:K])
            y_k = ladder(ky[:K])
            w_b = MIN_W + ladder(cwk)
            h_b = MIN_H + ladder(chk)
            d_k = MIN_D + _softplus(ladder(u[:K]))
            d_k1 = MIN_D + _softplus(ladder(u, shift=1))

            inv_w = 1.0 / w_b
            xi = (x - x_k) * inv_w
            s_k = h_b * inv_w
            omx = 1.0 - xi
            ximx = xi * omx
            xi2 = xi * xi
            num = h_b * (s_k * xi2 + d_k * ximx)
            den = s_k + (d_k1 - s_k + d_k - s_k) * ximx
            r_den = 1.0 / den
            y = y_k + num * r_den
            num_g = (s_k * s_k) * (d_k1 * xi2 + (s_k + s_k) * ximx
                                   + d_k * omx * omx)
            dy_dx = num_g * r_den * r_den * inv_w
            log_det = _plog(jnp.abs(dy_dx))

            outside = (x <= -TAIL) | (x >= TAIL)
            outv[slot, sl] = jnp.where(outside, x, y)
            ldv[sl] = ldv[sl] + jnp.where(outside, 0.0, log_det)

        pltpu.async_copy(outv.at[slot], out_hbm.at[d, bsl], out_sems[slot])

        @pl.when(d + 2 < D)
        def _():
            issue_in(d + 2, slot)

    def col_pair(dp, _):
        process(dp * 2, 0)
        process(dp * 2 + 1, 1)
        return 0

    lax.fori_loop(0, D // 2, col_pair, 0)

    pltpu.make_async_copy(outv.at[0], out_hbm.at[D - 2, bsl], out_sem0).wait()
    pltpu.make_async_copy(outv.at[1], out_hbm.at[D - 1, bsl], out_sem1).wait()
    pltpu.sync_copy(ldv, ld_hbm.at[bsl])


@jax.jit
def _run(x, uw, uh, ud):
    mesh = plsc.VectorSubcoreMesh(core_axis_name="c", subcore_axis_name="s")
    f = pl.kernel(
        _sc_body,
        mesh=mesh,
        compiler_params=pltpu.CompilerParams(needs_layout_passes=False),
        out_type=(
            jax.ShapeDtypeStruct((D, B), jnp.float32),
            jax.ShapeDtypeStruct((B,), jnp.float32),
        ),
        scratch_types=[
            pltpu.VMEM((2, BW), jnp.float32),
            pltpu.VMEM((2, K, BW), jnp.float32),
            pltpu.VMEM((2, K, BW), jnp.float32),
            pltpu.VMEM((2, K + 1, BW), jnp.float32),
            pltpu.VMEM((2, BW), jnp.float32),
            pltpu.VMEM((BW,), jnp.float32),
            pltpu.SemaphoreType.DMA,
            pltpu.SemaphoreType.DMA,
            pltpu.SemaphoreType.DMA,
            pltpu.SemaphoreType.DMA,
        ],
    )
    out_t, ld = f(x.T, uw.transpose(1, 2, 0), uh.transpose(1, 2, 0),
                  ud.transpose(2, 1, 0))
    return out_t.T, ld


def kernel(x, unnormalized_widths, unnormalized_heights, unnormalized_derivatives):
    return _run(x, unnormalized_widths, unnormalized_heights,
                unnormalized_derivatives)
